# baseline (speedup 1.0000x reference)
import functools

import jax
import jax.numpy as jnp
import numpy as np
from jax import lax
from jax.experimental import pallas as pl
from jax.experimental.pallas import tpu as pltpu
from jax.experimental.pallas import tpu_sc as plsc

_N_NODES = 100000
_N_DIH = 3200000
_N_TYPES = 16
_N_DEG = 5
_NB = 64

_NC = 2
_NS = 16
_L = 16
_NW = _NC * _NS
_W = _N_DIH // _NW
_C = 128
_NCHUNK = _W // _C
_GRP = _C // _L
_TAIL = _W - _NCHUNK * _C
_TAIL_GRP_LO = _GRP - _TAIL // _L

_MAGIC = np.int32(0x5F3759DF)


def _rsqrt_fast(x):
    y = plsc.bitcast(_MAGIC - jnp.right_shift(plsc.bitcast(x, jnp.int32), 1),
                     jnp.float32)
    for _ in range(3):
        y = y * (1.5 - 0.5 * x * y * y)
    return y


def _sc_body(ptab, ktab, mapg, mbat, out, *scr):
    mbufs = (scr[0:4], scr[4:8])
    bidb = (scr[8], scr[9])
    pbufs = (scr[10:14], scr[14:18])
    ttv, krows, ctv, stv, acc, kq = scr[18:24]
    semM, semP0, semP1, semK = scr[24:28]
    semPs = (semP0, semP1)

    wid = lax.axis_index("c") * _NS + lax.axis_index("s")
    base_w = wid * _W
    lane = lax.iota(jnp.int32, _L)

    zero16 = jnp.zeros((_L,), jnp.float32)
    for i in range(_NB):
        acc[pl.ds(i * _L, _L)] = zero16

    def _col(c):
        return jnp.full((_L,), c, jnp.int32)

    def fire_map(base, b):
        for j in range(4):
            pltpu.async_copy(mapg.at[j, pl.ds(base, _C)], mbufs[b][j], semM)
        pltpu.async_copy(mbat.at[pl.ds(base, _C)], bidb[b], semM)

    def drain_map(b):
        for j in range(4):
            pltpu.make_async_copy(mapg.at[j, pl.ds(base_w, _C)],
                                  mbufs[b][j], semM).wait()
        pltpu.make_async_copy(mbat.at[pl.ds(base_w, _C)], bidb[b], semM).wait()

    def fire_pos(b):
        for j in range(4):
            pltpu.async_copy(ptab.at[mbufs[b][j]], pbufs[b][j], semPs[b])

    def drain_pos(b):
        for j in range(4):
            pltpu.make_async_copy(ptab.at[mbufs[b][j]], pbufs[b][j],
                                  semPs[b]).wait()

    def pass1_tt(b, grp_lo):
        prefs = pbufs[b]
        for g in range(grp_lo, _GRP):
            ridx = lane + g * _L
            t = [plsc.bitcast(plsc.load_gather(prefs[j], [ridx, _col(3)]),
                              jnp.int32) for j in range(4)]
            tt = ((t[0] * _N_TYPES + t[1]) * _N_TYPES + t[2]) * _N_TYPES + t[3]
            ttv[pl.ds(g * _L, _L)] = tt

    def pass_geom(b, grp_lo):
        prefs = pbufs[b]
        for g in range(grp_lo, _GRP):
            ridx = lane + g * _L
            px = [plsc.load_gather(prefs[j], [ridx, _col(0)]) for j in range(4)]
            py = [plsc.load_gather(prefs[j], [ridx, _col(1)]) for j in range(4)]
            pz = [plsc.load_gather(prefs[j], [ridx, _col(2)]) for j in range(4)]
            b1x, b1y, b1z = px[1] - px[0], py[1] - py[0], pz[1] - pz[0]
            b2x, b2y, b2z = px[2] - px[1], py[2] - py[1], pz[2] - pz[1]
            b3x, b3y, b3z = px[3] - px[2], py[3] - py[2], pz[3] - pz[2]
            n1x = b1y * b2z - b1z * b2y
            n1y = b1z * b2x - b1x * b2z
            n1z = b1x * b2y - b1y * b2x
            n2x = b2y * b3z - b2z * b3y
            n2y = b2z * b3x - b2x * b3z
            n2z = b2x * b3y - b2y * b3x
            xd = n1x * n2x + n1y * n2y + n1z * n2z
            s = b2x * b2x + b2y * b2y + b2z * b2z
            b2n = s * _rsqrt_fast(s)
            inv_den = 1.0 / (b2n + 1e-12)
            cx = n1y * b2z - n1z * b2y
            cy = n1z * b2x - n1x * b2z
            cz = n1x * b2y - n1y * b2x
            yd = (cx * n2x + cy * n2y + cz * n2z) * inv_den
            r2 = xd * xd + yd * yd
            inv_r = _rsqrt_fast(r2)
            degen = r2 == 0.0
            ctv[pl.ds(g * _L, _L)] = jnp.where(degen, 1.0, xd * inv_r)
            stv[pl.ds(g * _L, _L)] = jnp.where(degen, 0.0, yd * inv_r)

    l129 = lane * 129

    def transpose_k(grp_lo):
        for d in range(grp_lo * _L, _C):
            plsc.store_scatter(kq, [l129 + d], krows[d, :])

    def pass_energy(b, grp_lo):
        for g in range(grp_lo, _GRP):
            ct = ctv[pl.ds(g * _L, _L)]
            st = stv[pl.ds(g * _L, _L)]
            goff = g * _L

            def _kc(c):
                return kq[pl.ds(c * 129 + goff, _L)]

            v = _kc(5)
            v = v + _kc(1) * st + _kc(6) * ct
            ck, sk = ct, st
            for d in range(2, _N_DEG):
                ck, sk = ck * ct - sk * st, sk * ct + ck * st
                v = v + _kc(d) * sk + _kc(5 + d) * ck
            bid = bidb[b][pl.ds(g * _L, _L)]
            plsc.addupdate_scatter(acc, [bid * _L + lane], v)

    def consume_chunk(i, b, prefetch):
        if prefetch:
            fire_map(base_w + (i + 1) * _C, 1 - b)
        drain_pos(b)
        pass1_tt(b, 0)
        kcp = pltpu.async_copy(ktab.at[ttv], krows, semK)
        pass_geom(b, 0)
        if prefetch:
            drain_map(1 - b)
            fire_pos(1 - b)
        kcp.wait()
        transpose_k(0)
        pass_energy(b, 0)

    fire_map(base_w, 0)
    drain_map(0)
    fire_pos(0)

    def _pair(t, carry):
        consume_chunk(2 * t, 0, True)
        consume_chunk(2 * t + 1, 1, True)
        return carry

    lax.fori_loop(0, (_NCHUNK - 1) // 2, _pair, 0)
    consume_chunk(_NCHUNK - 1, 0, False)

    tbase = base_w + _W - _C
    fire_map(tbase, 1)
    drain_map(1)
    fire_pos(1)
    drain_pos(1)
    pass1_tt(1, _TAIL_GRP_LO)
    kcp = pltpu.async_copy(ktab.at[ttv], krows, semK)
    pass_geom(1, _TAIL_GRP_LO)
    kcp.wait()
    transpose_k(_TAIL_GRP_LO)
    pass_energy(1, _TAIL_GRP_LO)

    pltpu.sync_copy(acc, out.at[wid])


_dihedral_sc = functools.partial(
    pl.kernel,
    out_type=jax.ShapeDtypeStruct((_NW, _NB * _L), jnp.float32),
    mesh=plsc.VectorSubcoreMesh(core_axis_name="c", subcore_axis_name="s",
                                num_cores=_NC, num_subcores=_NS),
    compiler_params=pltpu.CompilerParams(needs_layout_passes=False,
                                         use_tc_tiling_on_sc=False),
    scratch_types=(
        [pltpu.VMEM((_C,), jnp.int32) for _ in range(8)]
        + [pltpu.VMEM((_C,), jnp.int32) for _ in range(2)]
        + [pltpu.VMEM((_C, 16), jnp.float32) for _ in range(8)]
        + [pltpu.VMEM((_C,), jnp.int32),
           pltpu.VMEM((_C, 16), jnp.float32),
           pltpu.VMEM((_C,), jnp.float32),
           pltpu.VMEM((_C,), jnp.float32),
           pltpu.VMEM((_NB * _L,), jnp.float32),
           pltpu.VMEM((16 * 129,), jnp.float32)]
        + [pltpu.SemaphoreType.DMA for _ in range(4)]
    ),
)(_sc_body)


def _ptab_pack_body(pos_ref, atf_ref, out_ref):
    n = pos_ref.shape[0]
    out_ref[...] = jnp.concatenate(
        [pos_ref[...], atf_ref[...], jnp.zeros((n, 12), jnp.float32)], axis=1)


def _ktab_pack_body(a_ref, b_ref, out_ref):
    n = a_ref.shape[0]
    out_ref[...] = jnp.concatenate(
        [a_ref[...], b_ref[...], jnp.zeros((n, 6), jnp.float32)], axis=1)


_ptab_pack = pl.pallas_call(
    _ptab_pack_body,
    grid=(_N_NODES // 1000,),
    in_specs=[pl.BlockSpec((1000, 3), lambda i: (i, 0)),
              pl.BlockSpec((1000, 1), lambda i: (i, 0))],
    out_specs=pl.BlockSpec((1000, 16), lambda i: (i, 0)),
    out_shape=jax.ShapeDtypeStruct((_N_NODES, 16), jnp.float32),
)

_NT4 = _N_TYPES ** 4
_ktab_pack = pl.pallas_call(
    _ktab_pack_body,
    grid=(_NT4 // 4096,),
    in_specs=[pl.BlockSpec((4096, _N_DEG), lambda i: (i, 0)),
              pl.BlockSpec((4096, _N_DEG), lambda i: (i, 0))],
    out_specs=pl.BlockSpec((4096, 16), lambda i: (i, 0)),
    out_shape=jax.ShapeDtypeStruct((_NT4, 16), jnp.float32),
)


def kernel(pos, k1s, k2s, mapping, mapping_batch, atom_types):
    map32 = mapping.astype(jnp.int32)
    mb32 = mapping_batch.astype(jnp.int32)
    at32 = atom_types.astype(jnp.int32)
    atf = lax.bitcast_convert_type(at32, jnp.float32).reshape(-1, 1)
    ptab = _ptab_pack(pos.astype(jnp.float32), atf)
    k1f = jnp.transpose(k1s.reshape(_N_DEG, -1))
    k2f = jnp.transpose(k2s.reshape(_N_DEG, -1))
    ktab = _ktab_pack(k1f, k2f)
    parts = _dihedral_sc(ptab, ktab, map32, mb32)
    return parts.reshape(_NW, _NB, _L).sum(axis=(0, 2))


# speedup vs baseline: 1.0298x; 1.0298x over previous
import functools

import jax
import jax.numpy as jnp
import numpy as np
from jax import lax
from jax.experimental import pallas as pl
from jax.experimental.pallas import tpu as pltpu
from jax.experimental.pallas import tpu_sc as plsc

_N_NODES = 100000
_N_DIH = 3200000
_N_TYPES = 16
_N_DEG = 5
_NB = 64

_NC = 2
_NS = 16
_L = 16
_NW = _NC * _NS
_W = _N_DIH // _NW
_C = 128
_NCHUNK = _W // _C
_GRP = _C // _L
_TAIL = _W - _NCHUNK * _C
_TAIL_GRP_LO = _GRP - _TAIL // _L

_MAGIC = np.int32(0x5F3759DF)


def _rsqrt_fast(x):
    y = plsc.bitcast(_MAGIC - jnp.right_shift(plsc.bitcast(x, jnp.int32), 1),
                     jnp.float32)
    for _ in range(3):
        y = y * (1.5 - 0.5 * x * y * y)
    return y


def _sc_body(ptab, ktab, mapg, mbat, out, *scr):
    mbufs = (scr[0:4], scr[4:8])
    bidb = (scr[8], scr[9])
    pbufs = (scr[10:14], scr[14:18])
    ttv, krows, ctv, stv, acc = scr[18:23]
    semM, semP0, semP1, semK = scr[23:27]
    semPs = (semP0, semP1)

    wid = lax.axis_index("c") * _NS + lax.axis_index("s")
    base_w = wid * _W
    lane = lax.iota(jnp.int32, _L)

    zero16 = jnp.zeros((_L,), jnp.float32)
    for i in range(_NB):
        acc[pl.ds(i * _L, _L)] = zero16

    def _col(c):
        return jnp.full((_L,), c, jnp.int32)

    def fire_map(base, b):
        for j in range(4):
            pltpu.async_copy(mapg.at[j, pl.ds(base, _C)], mbufs[b][j], semM)
        pltpu.async_copy(mbat.at[pl.ds(base, _C)], bidb[b], semM)

    def drain_map(b):
        for j in range(4):
            pltpu.make_async_copy(mapg.at[j, pl.ds(base_w, _C)],
                                  mbufs[b][j], semM).wait()
        pltpu.make_async_copy(mbat.at[pl.ds(base_w, _C)], bidb[b], semM).wait()

    def fire_pos(b):
        for j in range(4):
            pltpu.async_copy(ptab.at[mbufs[b][j]], pbufs[b][j], semPs[b])

    def drain_pos(b):
        for j in range(4):
            pltpu.make_async_copy(ptab.at[mbufs[b][j]], pbufs[b][j],
                                  semPs[b]).wait()

    def pass1_tt(b, grp_lo):
        prefs = pbufs[b]
        for g in range(grp_lo, _GRP):
            ridx = lane + g * _L
            t = [plsc.bitcast(plsc.load_gather(prefs[j], [ridx, _col(3)]),
                              jnp.int32) for j in range(4)]
            tt = ((t[0] * _N_TYPES + t[1]) * _N_TYPES + t[2]) * _N_TYPES + t[3]
            ttv[pl.ds(g * _L, _L)] = tt

    def pass_geom(b, grp_lo):
        prefs = pbufs[b]
        for g in range(grp_lo, _GRP):
            ridx = lane + g * _L
            px = [plsc.load_gather(prefs[j], [ridx, _col(0)]) for j in range(4)]
            py = [plsc.load_gather(prefs[j], [ridx, _col(1)]) for j in range(4)]
            pz = [plsc.load_gather(prefs[j], [ridx, _col(2)]) for j in range(4)]
            b1x, b1y, b1z = px[1] - px[0], py[1] - py[0], pz[1] - pz[0]
            b2x, b2y, b2z = px[2] - px[1], py[2] - py[1], pz[2] - pz[1]
            b3x, b3y, b3z = px[3] - px[2], py[3] - py[2], pz[3] - pz[2]
            n1x = b1y * b2z - b1z * b2y
            n1y = b1z * b2x - b1x * b2z
            n1z = b1x * b2y - b1y * b2x
            n2x = b2y * b3z - b2z * b3y
            n2y = b2z * b3x - b2x * b3z
            n2z = b2x * b3y - b2y * b3x
            xd = n1x * n2x + n1y * n2y + n1z * n2z
            s = b2x * b2x + b2y * b2y + b2z * b2z
            b2n = s * _rsqrt_fast(s)
            inv_den = 1.0 / (b2n + 1e-12)
            cx = n1y * b2z - n1z * b2y
            cy = n1z * b2x - n1x * b2z
            cz = n1x * b2y - n1y * b2x
            yd = (cx * n2x + cy * n2y + cz * n2z) * inv_den
            r2 = xd * xd + yd * yd
            inv_r = _rsqrt_fast(r2)
            degen = r2 == 0.0
            ctv[pl.ds(g * _L, _L)] = jnp.where(degen, 1.0, xd * inv_r)
            stv[pl.ds(g * _L, _L)] = jnp.where(degen, 0.0, yd * inv_r)

    def pass_energy(b, grp_lo):
        for g in range(grp_lo, _GRP):
            ridx = lane + g * _L
            ct = ctv[pl.ds(g * _L, _L)]
            st = stv[pl.ds(g * _L, _L)]

            def _kc(c):
                return plsc.load_gather(krows, [ridx, _col(c)])

            v = _kc(5)
            v = v + _kc(1) * st + _kc(6) * ct
            ck, sk = ct, st
            for d in range(2, _N_DEG):
                ck, sk = ck * ct - sk * st, sk * ct + ck * st
                v = v + _kc(d) * sk + _kc(5 + d) * ck
            bid = bidb[b][pl.ds(g * _L, _L)]
            plsc.addupdate_scatter(acc, [bid * _L + lane], v)

    def consume_chunk(i, b, prefetch):
        if prefetch:
            fire_map(base_w + (i + 1) * _C, 1 - b)
        drain_pos(b)
        pass1_tt(b, 0)
        kcp = pltpu.async_copy(ktab.at[ttv], krows, semK)
        pass_geom(b, 0)
        if prefetch:
            drain_map(1 - b)
            fire_pos(1 - b)
        kcp.wait()
        pass_energy(b, 0)

    fire_map(base_w, 0)
    drain_map(0)
    fire_pos(0)

    def _pair(t, carry):
        consume_chunk(2 * t, 0, True)
        consume_chunk(2 * t + 1, 1, True)
        return carry

    lax.fori_loop(0, (_NCHUNK - 1) // 2, _pair, 0)
    consume_chunk(_NCHUNK - 1, 0, False)

    tbase = base_w + _W - _C
    fire_map(tbase, 1)
    drain_map(1)
    fire_pos(1)
    drain_pos(1)
    pass1_tt(1, _TAIL_GRP_LO)
    kcp = pltpu.async_copy(ktab.at[ttv], krows, semK)
    pass_geom(1, _TAIL_GRP_LO)
    kcp.wait()
    pass_energy(1, _TAIL_GRP_LO)

    pltpu.sync_copy(acc, out.at[wid])


_dihedral_sc = functools.partial(
    pl.kernel,
    out_type=jax.ShapeDtypeStruct((_NW, _NB * _L), jnp.float32),
    mesh=plsc.VectorSubcoreMesh(core_axis_name="c", subcore_axis_name="s",
                                num_cores=_NC, num_subcores=_NS),
    compiler_params=pltpu.CompilerParams(needs_layout_passes=False,
                                         use_tc_tiling_on_sc=False),
    scratch_types=(
        [pltpu.VMEM((_C,), jnp.int32) for _ in range(8)]
        + [pltpu.VMEM((_C,), jnp.int32) for _ in range(2)]
        + [pltpu.VMEM((_C, 16), jnp.float32) for _ in range(8)]
        + [pltpu.VMEM((_C,), jnp.int32),
           pltpu.VMEM((_C, 16), jnp.float32),
           pltpu.VMEM((_C,), jnp.float32),
           pltpu.VMEM((_C,), jnp.float32),
           pltpu.VMEM((_NB * _L,), jnp.float32)]
        + [pltpu.SemaphoreType.DMA for _ in range(4)]
    ),
)(_sc_body)


def _ptab_pack_body(pos_ref, atf_ref, out_ref):
    n = pos_ref.shape[0]
    out_ref[...] = jnp.concatenate(
        [pos_ref[...], atf_ref[...], jnp.zeros((n, 12), jnp.float32)], axis=1)


def _ktab_pack_body(a_ref, b_ref, out_ref):
    n = a_ref.shape[0]
    out_ref[...] = jnp.concatenate(
        [a_ref[...], b_ref[...], jnp.zeros((n, 6), jnp.float32)], axis=1)


_ptab_pack = pl.pallas_call(
    _ptab_pack_body,
    grid=(_N_NODES // 1000,),
    in_specs=[pl.BlockSpec((1000, 3), lambda i: (i, 0)),
              pl.BlockSpec((1000, 1), lambda i: (i, 0))],
    out_specs=pl.BlockSpec((1000, 16), lambda i: (i, 0)),
    out_shape=jax.ShapeDtypeStruct((_N_NODES, 16), jnp.float32),
)

_NT4 = _N_TYPES ** 4
_ktab_pack = pl.pallas_call(
    _ktab_pack_body,
    grid=(_NT4 // 4096,),
    in_specs=[pl.BlockSpec((4096, _N_DEG), lambda i: (i, 0)),
              pl.BlockSpec((4096, _N_DEG), lambda i: (i, 0))],
    out_specs=pl.BlockSpec((4096, 16), lambda i: (i, 0)),
    out_shape=jax.ShapeDtypeStruct((_NT4, 16), jnp.float32),
)


def kernel(pos, k1s, k2s, mapping, mapping_batch, atom_types):
    map32 = mapping.astype(jnp.int32)
    mb32 = mapping_batch.astype(jnp.int32)
    at32 = atom_types.astype(jnp.int32)
    atf = lax.bitcast_convert_type(at32, jnp.float32).reshape(-1, 1)
    ptab = _ptab_pack(pos.astype(jnp.float32), atf)
    k1f = jnp.transpose(k1s.reshape(_N_DEG, -1))
    k2f = jnp.transpose(k2s.reshape(_N_DEG, -1))
    ktab = _ktab_pack(k1f, k2f)
    parts = _dihedral_sc(ptab, ktab, map32, mb32)
    return parts.reshape(_NW, _NB, _L).sum(axis=(0, 2))
